# baseline (speedup 1.0000x reference)
import functools

import jax
import jax.numpy as jnp
from jax import lax
from jax.experimental import pallas as pl
from jax.experimental.pallas import tpu as pltpu
from jax.experimental.pallas import tpu_sc as plsc

G = 64
N = 50000
D = 256
LANES = 16
NUM_TILES = 16
BINS = 128
NACC = 4

ROW_BLOCK = 6144
N_PAD = 55296

CHUNK = 3136
TAIL = N - 15 * CHUNK


def _rowsum_body(p_ref, t_ref, o_ref):
    o_ref[...] = jnp.sum(jnp.abs(p_ref[...] - t_ref[...]), axis=1)


def _per_node_sums(pred, target):
    d = pred.shape[1]
    grid = N_PAD // ROW_BLOCK
    return pl.pallas_call(
        _rowsum_body,
        grid=(grid,),
        in_specs=[
            pl.BlockSpec((ROW_BLOCK, d), lambda i: (i, 0)),
            pl.BlockSpec((ROW_BLOCK, d), lambda i: (i, 0)),
        ],
        out_specs=pl.BlockSpec((ROW_BLOCK,), lambda i: (i,)),
        out_shape=jax.ShapeDtypeStruct((N_PAD,), jnp.float32),
    )(pred, target)


def _zero_accs(acc):
    zeros = jnp.zeros((LANES,), jnp.float32)
    for k in range(NACC):
        for j in range(BINS // LANES):
            acc[k, pl.ds(j * LANES, LANES)] = zeros


def _scatter_rotating(vals_v, ids_v, acc, count, with_vals):
    ones = jnp.ones((LANES,), jnp.float32)
    groups = count // (NACC * LANES)
    rem = (count - groups * NACC * LANES) // LANES

    def body(i, carry):
        base = i * (NACC * LANES)
        for k in range(NACC):
            sl = pl.ds(base + k * LANES, LANES)
            ids = ids_v[sl]
            v = vals_v[sl] if with_vals else ones
            plsc.addupdate_scatter(acc.at[k], [ids], v)
        return carry

    lax.fori_loop(0, groups, body, 0, unroll=2)
    for k in range(rem):
        sl = pl.ds(groups * NACC * LANES + k * LANES, LANES)
        ids = ids_v[sl]
        v = vals_v[sl] if with_vals else ones
        plsc.addupdate_scatter(acc.at[k], [ids], v)


def _merge_accs(acc, out_ref):
    for j in range(BINS // LANES):
        sl = pl.ds(j * LANES, LANES)
        s = acc[0, sl]
        for k in range(1, NACC):
            s = s + acc[k, sl]
        out_ref[sl] = s


@functools.cache
def _make_counts():
    mesh = plsc.VectorSubcoreMesh(core_axis_name="c", subcore_axis_name="s")

    @functools.partial(
        pl.kernel,
        out_type=jax.ShapeDtypeStruct((BINS,), jnp.float32),
        mesh=mesh,
        scratch_types=[
            pltpu.VMEM((CHUNK,), jnp.int32),
            pltpu.VMEM((NACC, BINS), jnp.float32),
            pltpu.VMEM((BINS,), jnp.float32),
            pltpu.VMEM_SHARED((NUM_TILES, BINS), jnp.float32),
            pltpu.VMEM((NUM_TILES, BINS), jnp.float32),
        ],
        compiler_params=pltpu.CompilerParams(needs_layout_passes=False),
    )
    def _counts(ids_hbm, out_hbm, ids_v, acc, acc_m, slab, slab_v):
        cid = lax.axis_index("c")
        sid = lax.axis_index("s")

        @pl.when(cid == 0)
        def _():
            def count_chunk(count):
                pltpu.sync_copy(
                    ids_hbm.at[pl.ds(sid * CHUNK, count)],
                    ids_v.at[pl.ds(0, count)])
                _zero_accs(acc)
                _scatter_rotating(None, ids_v, acc, count, False)
                _merge_accs(acc, acc_m)

            @pl.when(sid < NUM_TILES - 1)
            def _():
                count_chunk(CHUNK)

            @pl.when(sid == NUM_TILES - 1)
            def _():
                count_chunk(TAIL)

            pltpu.sync_copy(acc_m, slab.at[sid])
            plsc.subcore_barrier()

            @pl.when(sid == 0)
            def _():
                pltpu.sync_copy(slab, slab_v)
                for j in range(BINS // LANES):
                    sl = pl.ds(j * LANES, LANES)
                    c = slab_v[0, sl]
                    for t in range(1, NUM_TILES):
                        c = c + slab_v[t, sl]
                    acc_m[sl] = c
                pltpu.sync_copy(acc_m, out_hbm)

    return _counts


@functools.cache
def _make_segment_mean():
    mesh = plsc.VectorSubcoreMesh(core_axis_name="c", subcore_axis_name="s")

    @functools.partial(
        pl.kernel,
        out_type=jax.ShapeDtypeStruct((1,), jnp.float32),
        mesh=mesh,
        scratch_types=[
            pltpu.VMEM((CHUNK,), jnp.float32),
            pltpu.VMEM((CHUNK,), jnp.int32),
            pltpu.VMEM((NACC, BINS), jnp.float32),
            pltpu.VMEM((BINS,), jnp.float32),
            pltpu.VMEM((BINS,), jnp.float32),
            pltpu.VMEM_SHARED((NUM_TILES, BINS), jnp.float32),
            pltpu.VMEM((NUM_TILES, BINS), jnp.float32),
            pltpu.VMEM((LANES,), jnp.float32),
            pltpu.SemaphoreType.DMA,
            pltpu.SemaphoreType.DMA,
        ],
        compiler_params=pltpu.CompilerParams(needs_layout_passes=False),
    )
    def _segment_mean(vals_hbm, ids_hbm, cnt_hbm, out_hbm,
                      vals_v, ids_v, acc, acc_m, cnt_v, slab, slab_v, out_v,
                      sem_a, sem_b):
        cid = lax.axis_index("c")
        sid = lax.axis_index("s")

        @pl.when(cid == 0)
        def _():
            def scatter_chunk(count):
                base = sid * CHUNK
                cp_v = pltpu.async_copy(
                    vals_hbm.at[pl.ds(base, count)],
                    vals_v.at[pl.ds(0, count)], sem_a)
                cp_i = pltpu.async_copy(
                    ids_hbm.at[pl.ds(base, count)],
                    ids_v.at[pl.ds(0, count)], sem_b)
                _zero_accs(acc)
                cp_v.wait()
                cp_i.wait()
                _scatter_rotating(vals_v, ids_v, acc, count, True)
                _merge_accs(acc, acc_m)

            @pl.when(sid < NUM_TILES - 1)
            def _():
                scatter_chunk(CHUNK)

            @pl.when(sid == NUM_TILES - 1)
            def _():
                scatter_chunk(TAIL)

            pltpu.sync_copy(acc_m, slab.at[sid])
            plsc.subcore_barrier()

            @pl.when(sid == 0)
            def _():
                cp_c = pltpu.async_copy(cnt_hbm, cnt_v, sem_a)
                pltpu.sync_copy(slab, slab_v)
                cp_c.wait()

                acc_f = jnp.zeros((LANES,), jnp.float32)
                for j in range(G // LANES):
                    sl = pl.ds(j * LANES, LANES)
                    s = slab_v[0, sl]
                    for t in range(1, NUM_TILES):
                        s = s + slab_v[t, sl]
                    c = cnt_v[sl]
                    acc_f = acc_f + s / (c * float(D))
                res = jnp.sum(acc_f) * (10000.0 / float(G))
                out_v[...] = jnp.broadcast_to(res, (LANES,))
                pltpu.sync_copy(out_v.at[pl.ds(0, 1)], out_hbm)

    return _segment_mean


def kernel(pred, target, batch, x):
    ids = batch.astype(jnp.int32)
    counts = _make_counts()(ids)
    per_node = _per_node_sums(pred, target)
    out = _make_segment_mean()(per_node, ids, counts)
    return out[0]


# speedup vs baseline: 1.0357x; 1.0357x over previous
import functools

import jax
import jax.numpy as jnp
from jax import lax
from jax.experimental import pallas as pl
from jax.experimental.pallas import tpu as pltpu
from jax.experimental.pallas import tpu_sc as plsc

G = 64
N = 50000
D = 256
LANES = 16
NUM_TILES = 16
BINS = 128
NACC = 4

ROW_BLOCK = 5120
N_PAD = 51200

CHUNK = 3136
TAIL = N - 15 * CHUNK


def _rowsum_body(p_ref, t_ref, o_ref):
    o_ref[...] = jnp.sum(jnp.abs(p_ref[...] - t_ref[...]), axis=1)


def _per_node_sums(pred, target):
    d = pred.shape[1]
    grid = N_PAD // ROW_BLOCK
    return pl.pallas_call(
        _rowsum_body,
        grid=(grid,),
        in_specs=[
            pl.BlockSpec((ROW_BLOCK, d), lambda i: (i, 0)),
            pl.BlockSpec((ROW_BLOCK, d), lambda i: (i, 0)),
        ],
        out_specs=pl.BlockSpec((ROW_BLOCK,), lambda i: (i,)),
        out_shape=jax.ShapeDtypeStruct((N_PAD,), jnp.float32),
    )(pred, target)


def _zero_accs(acc):
    zeros = jnp.zeros((LANES,), jnp.float32)
    for k in range(NACC):
        for j in range(BINS // LANES):
            acc[k, pl.ds(j * LANES, LANES)] = zeros


def _scatter_rotating(vals_v, ids_v, acc, count, with_vals):
    ones = jnp.ones((LANES,), jnp.float32)
    groups = count // (NACC * LANES)
    rem = (count - groups * NACC * LANES) // LANES

    def body(i, carry):
        base = i * (NACC * LANES)
        for k in range(NACC):
            sl = pl.ds(base + k * LANES, LANES)
            ids = ids_v[sl]
            v = vals_v[sl] if with_vals else ones
            plsc.addupdate_scatter(acc.at[k], [ids], v)
        return carry

    lax.fori_loop(0, groups, body, 0, unroll=2)
    for k in range(rem):
        sl = pl.ds(groups * NACC * LANES + k * LANES, LANES)
        ids = ids_v[sl]
        v = vals_v[sl] if with_vals else ones
        plsc.addupdate_scatter(acc.at[k], [ids], v)


def _merge_accs(acc, out_ref):
    for j in range(BINS // LANES):
        sl = pl.ds(j * LANES, LANES)
        s = acc[0, sl]
        for k in range(1, NACC):
            s = s + acc[k, sl]
        out_ref[sl] = s


@functools.cache
def _make_counts():
    mesh = plsc.VectorSubcoreMesh(core_axis_name="c", subcore_axis_name="s")

    @functools.partial(
        pl.kernel,
        out_type=jax.ShapeDtypeStruct((BINS,), jnp.float32),
        mesh=mesh,
        scratch_types=[
            pltpu.VMEM((CHUNK,), jnp.int32),
            pltpu.VMEM((NACC, BINS), jnp.float32),
            pltpu.VMEM((BINS,), jnp.float32),
            pltpu.VMEM_SHARED((NUM_TILES, BINS), jnp.float32),
            pltpu.VMEM((NUM_TILES, BINS), jnp.float32),
        ],
        compiler_params=pltpu.CompilerParams(needs_layout_passes=False),
    )
    def _counts(ids_hbm, out_hbm, ids_v, acc, acc_m, slab, slab_v):
        cid = lax.axis_index("c")
        sid = lax.axis_index("s")

        @pl.when(cid == 0)
        def _():
            def count_chunk(count):
                pltpu.sync_copy(
                    ids_hbm.at[pl.ds(sid * CHUNK, count)],
                    ids_v.at[pl.ds(0, count)])
                _zero_accs(acc)
                _scatter_rotating(None, ids_v, acc, count, False)
                _merge_accs(acc, acc_m)

            @pl.when(sid < NUM_TILES - 1)
            def _():
                count_chunk(CHUNK)

            @pl.when(sid == NUM_TILES - 1)
            def _():
                count_chunk(TAIL)

            pltpu.sync_copy(acc_m, slab.at[sid])
            plsc.subcore_barrier()

            @pl.when(sid == 0)
            def _():
                pltpu.sync_copy(slab, slab_v)
                for j in range(BINS // LANES):
                    sl = pl.ds(j * LANES, LANES)
                    c = slab_v[0, sl]
                    for t in range(1, NUM_TILES):
                        c = c + slab_v[t, sl]
                    acc_m[sl] = c
                pltpu.sync_copy(acc_m, out_hbm)

    return _counts


@functools.cache
def _make_segment_mean():
    mesh = plsc.VectorSubcoreMesh(core_axis_name="c", subcore_axis_name="s")

    @functools.partial(
        pl.kernel,
        out_type=jax.ShapeDtypeStruct((1,), jnp.float32),
        mesh=mesh,
        scratch_types=[
            pltpu.VMEM((CHUNK,), jnp.float32),
            pltpu.VMEM((CHUNK,), jnp.int32),
            pltpu.VMEM((NACC, BINS), jnp.float32),
            pltpu.VMEM((BINS,), jnp.float32),
            pltpu.VMEM((BINS,), jnp.float32),
            pltpu.VMEM_SHARED((NUM_TILES, BINS), jnp.float32),
            pltpu.VMEM((NUM_TILES, BINS), jnp.float32),
            pltpu.VMEM((LANES,), jnp.float32),
            pltpu.SemaphoreType.DMA,
            pltpu.SemaphoreType.DMA,
        ],
        compiler_params=pltpu.CompilerParams(needs_layout_passes=False),
    )
    def _segment_mean(vals_hbm, ids_hbm, cnt_hbm, out_hbm,
                      vals_v, ids_v, acc, acc_m, cnt_v, slab, slab_v, out_v,
                      sem_a, sem_b):
        cid = lax.axis_index("c")
        sid = lax.axis_index("s")

        @pl.when(cid == 0)
        def _():
            def scatter_chunk(count):
                base = sid * CHUNK
                cp_v = pltpu.async_copy(
                    vals_hbm.at[pl.ds(base, count)],
                    vals_v.at[pl.ds(0, count)], sem_a)
                cp_i = pltpu.async_copy(
                    ids_hbm.at[pl.ds(base, count)],
                    ids_v.at[pl.ds(0, count)], sem_b)
                _zero_accs(acc)
                cp_v.wait()
                cp_i.wait()
                _scatter_rotating(vals_v, ids_v, acc, count, True)
                _merge_accs(acc, acc_m)

            @pl.when(sid < NUM_TILES - 1)
            def _():
                scatter_chunk(CHUNK)

            @pl.when(sid == NUM_TILES - 1)
            def _():
                scatter_chunk(TAIL)

            pltpu.sync_copy(acc_m, slab.at[sid])
            plsc.subcore_barrier()

            @pl.when(sid == 0)
            def _():
                cp_c = pltpu.async_copy(cnt_hbm, cnt_v, sem_a)
                pltpu.sync_copy(slab, slab_v)
                cp_c.wait()

                acc_f = jnp.zeros((LANES,), jnp.float32)
                for j in range(G // LANES):
                    sl = pl.ds(j * LANES, LANES)
                    s = slab_v[0, sl]
                    for t in range(1, NUM_TILES):
                        s = s + slab_v[t, sl]
                    c = cnt_v[sl]
                    acc_f = acc_f + s / (c * float(D))
                res = jnp.sum(acc_f) * (10000.0 / float(G))
                out_v[...] = jnp.broadcast_to(res, (LANES,))
                pltpu.sync_copy(out_v.at[pl.ds(0, 1)], out_hbm)

    return _segment_mean


def kernel(pred, target, batch, x):
    ids = batch.astype(jnp.int32)
    counts = _make_counts()(ids)
    per_node = _per_node_sums(pred, target)
    out = _make_segment_mean()(per_node, ids, counts)
    return out[0]
